# baseline (device time: 89678 ns/iter reference)
import jax
import jax.numpy as jnp
from jax import lax
from jax.experimental import pallas as pl
from jax.experimental.pallas import tpu as pltpu

T = 512
D = 1024
V_SHARD = 8192
V = 2 * V_SHARD
CHUNK = 1024
NC = V_SHARD // CHUNK
NE = NC + 4


def kernel(x, W):
    def body(
        x_ref,
        w_ref,
        out_ref,
        logits_ref,
        e_ref,
        wbuf_ref,
        w_sems,
        sx_sems,
        rx_sems,
        sy_sems,
        ry_sems,
        sz_sems,
        rz_sems,
        out_sems,
    ):
        my_x = lax.axis_index("x")
        my_y = lax.axis_index("y")
        my_z = lax.axis_index("z")
        nbr_x = (1 - my_x, my_y, my_z)
        nbr_y = (my_x, 1 - my_y, my_z)
        nbr_z = (my_x, my_y, 1 - my_z)

        q_x = 2 * my_y + my_z
        q_y = 2 * (1 - my_y) + my_z

        def perm(j):
            return lax.rem(2 * q_x + j, NC)

        barrier = pltpu.get_barrier_semaphore()
        for nbr in (nbr_x, nbr_y, nbr_z):
            pl.semaphore_signal(
                barrier,
                inc=1,
                device_id=nbr,
                device_id_type=pl.DeviceIdType.MESH,
            )
        pl.semaphore_wait(barrier, 3)

        def w_dma(j):
            return pltpu.make_async_copy(
                w_ref.at[:, pl.ds(perm(j) * CHUNK, CHUNK)],
                wbuf_ref.at[j % 2],
                w_sems.at[j % 2],
            )

        def rdma(src_slot, dst_slot, send_sem, recv_sem, nbr):
            return pltpu.make_async_remote_copy(
                src_ref=logits_ref.at[src_slot],
                dst_ref=logits_ref.at[dst_slot],
                send_sem=send_sem,
                recv_sem=recv_sem,
                device_id=nbr,
                device_id_type=pl.DeviceIdType.MESH,
            )

        xb = x_ref[...].astype(jnp.bfloat16)

        w_dma(0).start()
        for j in range(NC):
            if j + 1 < NC:
                w_dma(j + 1).start()
            w_dma(j).wait()
            logits_ref[j] = jnp.dot(
                xb,
                wbuf_ref[j % 2].astype(jnp.bfloat16),
                preferred_element_type=jnp.float32,
            ).astype(jnp.bfloat16)
            if j < 2:
                rdma(
                    j, NC + 2 * q_x + j, sx_sems.at[j], rx_sems.at[j], nbr_x
                ).start()

        def chunk_exp(k, s_run, e_dst):
            e = jnp.exp(logits_ref[k].astype(jnp.float32))
            e_dst[...] = e.astype(jnp.bfloat16)
            return s_run + e.sum(axis=-1, keepdims=True)

        def mine_step(k, s_run):
            return chunk_exp(k, s_run, e_ref.at[k])

        s_run = lax.fori_loop(
            0, 4, mine_step, jnp.zeros((T, 1), jnp.float32)
        )
        for j in range(2):
            rdma(
                0, NC + 2 * q_x + j, sx_sems.at[j], rx_sems.at[j], nbr_x
            ).wait_recv()
            rdma(
                NC + 2 * q_x + j,
                NC + 2 * q_x + j,
                sy_sems.at[j],
                ry_sems.at[j],
                nbr_y,
            ).start()
            zi = 2 * my_y + j
            rdma(
                NC + 2 * q_x + j,
                NC + 2 * q_x + j,
                sz_sems.at[zi],
                rz_sems.at[zi],
                nbr_z,
            ).start()
        for j in range(2):
            s_run = lax.fori_loop(4 + 2 * j, 6 + 2 * j, mine_step, s_run)
            rdma(
                0, NC + 2 * q_y + j, sy_sems.at[j], ry_sems.at[j], nbr_y
            ).wait_recv()
            zi = 2 * (1 - my_y) + j
            rdma(
                NC + 2 * q_y + j,
                NC + 2 * q_y + j,
                sz_sems.at[zi],
                rz_sems.at[zi],
                nbr_z,
            ).start()

        def xq_step(j, s_run):
            return chunk_exp(NC + 2 * q_x + j, s_run, e_ref.at[NC + j])

        s_run = lax.fori_loop(0, 2, xq_step, s_run)

        def yq_step(j, s_run):
            return chunk_exp(NC + 2 * q_y + j, s_run, e_ref.at[NC + 2 + j])

        s_run = lax.fori_loop(0, 2, yq_step, s_run)

        def zq_step(i, s_run):
            c = 2 * (1 - my_z) + i + jnp.where(i >= 2, 2, 0)
            k = NC + c
            rdma(0, k, sz_sems.at[i], rz_sems.at[i], nbr_z).wait_recv()
            return chunk_exp(k, s_run, logits_ref.at[k])

        s = lax.fori_loop(0, 4, zq_step, s_run)
        r = 1.0 / s

        for j in range(2):
            rdma(j, NC, sx_sems.at[j], rx_sems.at[j], nbr_x).wait_send()
            rdma(j, NC, sy_sems.at[j], ry_sems.at[j], nbr_y).wait_send()
        for i in range(4):
            rdma(0, NC, sz_sems.at[i], rz_sems.at[i], nbr_z).wait_send()

        def scale_step(k, carry):
            c = k - NC
            quarter = lax.div(c, 2)
            e_idx = jnp.where(
                k < NC,
                k,
                jnp.where(
                    quarter == q_x,
                    NC + (c - 2 * q_x),
                    NC + 2 + (c - 2 * q_y),
                ),
            )
            in_e = jnp.logical_or(
                k < NC,
                jnp.logical_or(quarter == q_x, quarter == q_y),
            )

            @pl.when(in_e)
            def _():
                logits_ref[k] = (
                    e_ref[e_idx].astype(jnp.float32) * r
                ).astype(jnp.bfloat16)

            @pl.when(jnp.logical_not(in_e))
            def _():
                logits_ref[k] = (
                    logits_ref[k].astype(jnp.float32) * r
                ).astype(jnp.bfloat16)

            col_block = jnp.where(
                k < NC,
                my_x * NC + perm(k),
                (1 - my_x) * NC + c,
            )
            pltpu.make_async_copy(
                logits_ref.at[k],
                out_ref.at[:, pl.ds(col_block * CHUNK, CHUNK)],
                out_sems.at[k],
            ).start()
            return carry

        lax.fori_loop(0, 2 * NC, scale_step, 0)

        for j in range(2 * NC):
            col_block = jnp.where(
                j < NC,
                my_x * NC + perm(j),
                (1 - my_x) * NC + (j - NC),
            )
            pltpu.make_async_copy(
                logits_ref.at[j],
                out_ref.at[:, pl.ds(col_block * CHUNK, CHUNK)],
                out_sems.at[j],
            ).wait()

    return pl.pallas_call(
        body,
        out_shape=jax.ShapeDtypeStruct((T, V), jnp.bfloat16),
        in_specs=[
            pl.BlockSpec(memory_space=pltpu.VMEM),
            pl.BlockSpec(memory_space=pltpu.MemorySpace.HBM),
        ],
        out_specs=pl.BlockSpec(memory_space=pltpu.MemorySpace.HBM),
        scratch_shapes=[
            pltpu.VMEM((2 * NC, T, CHUNK), jnp.bfloat16),
            pltpu.VMEM((NE, T, CHUNK), jnp.bfloat16),
            pltpu.VMEM((2, D, CHUNK), jnp.float32),
            pltpu.SemaphoreType.DMA((2,)),
            pltpu.SemaphoreType.DMA((2,)),
            pltpu.SemaphoreType.DMA((2,)),
            pltpu.SemaphoreType.DMA((2,)),
            pltpu.SemaphoreType.DMA((2,)),
            pltpu.SemaphoreType.DMA((4,)),
            pltpu.SemaphoreType.DMA((4,)),
            pltpu.SemaphoreType.DMA((2 * NC,)),
        ],
        compiler_params=pltpu.CompilerParams(
            collective_id=0, vmem_limit_bytes=63 * 1024 * 1024
        ),
    )(x, W)


# device time: 89144 ns/iter; 1.0060x vs baseline; 1.0060x over previous
import jax
import jax.numpy as jnp
from jax import lax
from jax.experimental import pallas as pl
from jax.experimental.pallas import tpu as pltpu

T = 512
D = 1024
V_SHARD = 8192
V = 2 * V_SHARD
CHUNK = 1024
NC = V_SHARD // CHUNK


def kernel(x, W):
    def body(
        x_ref,
        w_ref,
        out_ref,
        e_slots,
        wbuf_ref,
        w_sems,
        sx_sems,
        rx_sems,
        sy_sems,
        ry_sems,
        sz_sems,
        rz_sems,
        out_sems,
    ):
        my_x = lax.axis_index("x")
        my_y = lax.axis_index("y")
        my_z = lax.axis_index("z")
        nbr_x = (1 - my_x, my_y, my_z)
        nbr_y = (my_x, 1 - my_y, my_z)
        nbr_z = (my_x, my_y, 1 - my_z)

        q_x = 2 * my_y + my_z
        q_y = 2 * (1 - my_y) + my_z

        def perm(j):
            return lax.rem(2 * q_x + j, NC)

        barrier = pltpu.get_barrier_semaphore()
        for nbr in (nbr_x, nbr_y, nbr_z):
            pl.semaphore_signal(
                barrier,
                inc=1,
                device_id=nbr,
                device_id_type=pl.DeviceIdType.MESH,
            )
        pl.semaphore_wait(barrier, 3)

        def w_dma(j):
            return pltpu.make_async_copy(
                w_ref.at[:, pl.ds(perm(j) * CHUNK, CHUNK)],
                wbuf_ref.at[j % 2],
                w_sems.at[j % 2],
            )

        def rdma(src_slot, dst_slot, send_sem, recv_sem, nbr):
            return pltpu.make_async_remote_copy(
                src_ref=e_slots.at[src_slot],
                dst_ref=e_slots.at[dst_slot],
                send_sem=send_sem,
                recv_sem=recv_sem,
                device_id=nbr,
                device_id_type=pl.DeviceIdType.MESH,
            )

        xb = x_ref[...].astype(jnp.bfloat16)

        s_run = jnp.zeros((T, 1), jnp.float32)
        w_dma(0).start()
        for j in range(NC):
            if j + 1 < NC:
                w_dma(j + 1).start()
            w_dma(j).wait()
            e = jnp.exp(
                jnp.dot(
                    xb,
                    wbuf_ref[j % 2].astype(jnp.bfloat16),
                    preferred_element_type=jnp.float32,
                )
            )
            e_slots[j] = e.astype(jnp.bfloat16)
            s_run = s_run + e.sum(axis=-1, keepdims=True)
            if j < 2:
                rdma(
                    j, NC + 2 * q_x + j, sx_sems.at[j], rx_sems.at[j], nbr_x
                ).start()

        def chunk_sum(k, s_run):
            return s_run + e_slots[k].astype(jnp.float32).sum(
                axis=-1, keepdims=True
            )

        for j in range(2):
            rdma(
                0, NC + 2 * q_x + j, sx_sems.at[j], rx_sems.at[j], nbr_x
            ).wait_recv()
            rdma(
                NC + 2 * q_x + j,
                NC + 2 * q_x + j,
                sy_sems.at[j],
                ry_sems.at[j],
                nbr_y,
            ).start()
            zi = 2 * my_y + j
            rdma(
                NC + 2 * q_x + j,
                NC + 2 * q_x + j,
                sz_sems.at[zi],
                rz_sems.at[zi],
                nbr_z,
            ).start()
            s_run = chunk_sum(NC + 2 * q_x + j, s_run)
        for j in range(2):
            rdma(
                0, NC + 2 * q_y + j, sy_sems.at[j], ry_sems.at[j], nbr_y
            ).wait_recv()
            zi = 2 * (1 - my_y) + j
            rdma(
                NC + 2 * q_y + j,
                NC + 2 * q_y + j,
                sz_sems.at[zi],
                rz_sems.at[zi],
                nbr_z,
            ).start()
            s_run = chunk_sum(NC + 2 * q_y + j, s_run)

        def zq_step(i, s_run):
            c = 2 * (1 - my_z) + i + jnp.where(i >= 2, 2, 0)
            k = NC + c
            rdma(0, k, sz_sems.at[i], rz_sems.at[i], nbr_z).wait_recv()
            return chunk_sum(k, s_run)

        s = lax.fori_loop(0, 4, zq_step, s_run)
        r = 1.0 / s

        for j in range(2):
            rdma(j, NC, sx_sems.at[j], rx_sems.at[j], nbr_x).wait_send()
            rdma(j, NC, sy_sems.at[j], ry_sems.at[j], nbr_y).wait_send()
        for i in range(4):
            rdma(0, NC, sz_sems.at[i], rz_sems.at[i], nbr_z).wait_send()

        def scale_step(k, carry):
            e_slots[k] = (e_slots[k].astype(jnp.float32) * r).astype(
                jnp.bfloat16
            )
            col_block = jnp.where(
                k < NC,
                my_x * NC + perm(k),
                (1 - my_x) * NC + (k - NC),
            )
            pltpu.make_async_copy(
                e_slots.at[k],
                out_ref.at[:, pl.ds(col_block * CHUNK, CHUNK)],
                out_sems.at[k],
            ).start()
            return carry

        lax.fori_loop(0, 2 * NC, scale_step, 0)

        for j in range(2 * NC):
            col_block = jnp.where(
                j < NC,
                my_x * NC + perm(j),
                (1 - my_x) * NC + (j - NC),
            )
            pltpu.make_async_copy(
                e_slots.at[j],
                out_ref.at[:, pl.ds(col_block * CHUNK, CHUNK)],
                out_sems.at[j],
            ).wait()

    return pl.pallas_call(
        body,
        out_shape=jax.ShapeDtypeStruct((T, V), jnp.bfloat16),
        in_specs=[
            pl.BlockSpec(memory_space=pltpu.VMEM),
            pl.BlockSpec(memory_space=pltpu.MemorySpace.HBM),
        ],
        out_specs=pl.BlockSpec(memory_space=pltpu.MemorySpace.HBM),
        scratch_shapes=[
            pltpu.VMEM((2 * NC, T, CHUNK), jnp.bfloat16),
            pltpu.VMEM((2, D, CHUNK), jnp.float32),
            pltpu.SemaphoreType.DMA((2,)),
            pltpu.SemaphoreType.DMA((2,)),
            pltpu.SemaphoreType.DMA((2,)),
            pltpu.SemaphoreType.DMA((2,)),
            pltpu.SemaphoreType.DMA((2,)),
            pltpu.SemaphoreType.DMA((4,)),
            pltpu.SemaphoreType.DMA((4,)),
            pltpu.SemaphoreType.DMA((2 * NC,)),
        ],
        compiler_params=pltpu.CompilerParams(
            collective_id=0, vmem_limit_bytes=63 * 1024 * 1024
        ),
    )(x, W)
